# Initial kernel scaffold; baseline (speedup 1.0000x reference)
import jax
import jax.numpy as jnp
from jax.experimental import pallas as pl


def kernel(x, y, edge_index, embedding, W, b):
    raise NotImplementedError("write your pallas kernel here")


# speedup vs baseline: 32.6957x; 32.6957x over previous
import functools

import jax
import jax.numpy as jnp
from jax import lax
from jax.experimental import pallas as pl
from jax.experimental.pallas import tpu as pltpu
from jax.experimental.pallas import tpu_sc as plsc

N = 10000
E = 320000
D = 128
B_IDX = 4096
LAMDA = 0.8

NC, NS, L = 2, 16, 16
NW = NC * NS
CHUNK = 128
BLK = 8
K = 80
NBLK = K // BLK
EP = NW * K * CHUNK
GN = 10112
PAD_ROWS = GN - N

_MESH = plsc.VectorSubcoreMesh(
    core_axis_name="c", subcore_axis_name="s", num_cores=NC, num_subcores=NS)

def _z16():
    return jnp.full((L,), 0.0, jnp.float32)


def _o16():
    return jnp.full((L,), 1.0, jnp.float32)


def _wid():
    return lax.axis_index("c") * NS + lax.axis_index("s")


@functools.partial(
    pl.kernel,
    out_type=jax.ShapeDtypeStruct((NC * GN,), jnp.float32),
    mesh=_MESH,
    scratch_types=[
        pltpu.VMEM((K, CHUNK), jnp.int32),
        pltpu.VMEM((640,), jnp.float32),
        pltpu.VMEM((CHUNK,), jnp.float32),
        pltpu.VMEM_SHARED((GN,), jnp.float32),
    ],
)
def _sc_degree(dst3, degp, idx_v, zbuf, ones_v, deg_sh):
    c = lax.axis_index("c")
    s = lax.axis_index("s")
    w = _wid()
    def fill_z(i, _):
        zbuf[pl.ds(i * L, L)] = _z16()
        return 0
    lax.fori_loop(0, 640 // L, fill_z, 0)
    def fill_o(i, _):
        ones_v[pl.ds(i * L, L)] = _o16()
        return 0
    lax.fori_loop(0, CHUNK // L, fill_o, 0)
    pltpu.sync_copy(zbuf.at[pl.ds(0, GN // NS)],
                    deg_sh.at[pl.ds(s * (GN // NS), GN // NS)])
    plsc.subcore_barrier()
    pltpu.sync_copy(dst3.at[w], idx_v)
    def chunk(j, _):
        pltpu.sync_copy(ones_v, deg_sh.at[idx_v.at[j]], add=True)
        return 0
    lax.fori_loop(0, K, chunk, 0)
    plsc.subcore_barrier()
    pltpu.sync_copy(deg_sh.at[pl.ds(s * (GN // NS), GN // NS)],
                    zbuf.at[pl.ds(0, GN // NS)])
    pltpu.sync_copy(zbuf.at[pl.ds(0, GN // NS)],
                    degp.at[pl.ds(c * GN + s * (GN // NS), GN // NS)])


def _tc_scale_body(emb, W, d0, d1, xw_ref, u_ref, dinv_ref):
    xw = jnp.dot(emb[...], W[...], preferred_element_type=jnp.float32)
    deg = d0[...] + d1[...] + 1.0
    dinv = lax.rsqrt(deg)
    xw_ref[...] = xw
    u_ref[...] = xw * dinv
    dinv_ref[...] = dinv


_R = 1000


def _tc_scale(emb, W, d0, d1):
    g = N // _R
    return pl.pallas_call(
        _tc_scale_body,
        grid=(g,),
        in_specs=[
            pl.BlockSpec((_R, D), lambda i: (i, 0)),
            pl.BlockSpec((D, D), lambda i: (0, 0)),
            pl.BlockSpec((_R, 1), lambda i: (i, 0)),
            pl.BlockSpec((_R, 1), lambda i: (i, 0)),
        ],
        out_specs=[
            pl.BlockSpec((_R, D), lambda i: (i, 0)),
            pl.BlockSpec((_R, D), lambda i: (i, 0)),
            pl.BlockSpec((_R, 1), lambda i: (i, 0)),
        ],
        out_shape=[
            jax.ShapeDtypeStruct((N, D), jnp.float32),
            jax.ShapeDtypeStruct((N, D), jnp.float32),
            jax.ShapeDtypeStruct((N, 1), jnp.float32),
        ],
    )(emb, W, d0, d1)


_ZROWS = GN // NS
_OROWS = N // NS


@functools.partial(
    pl.kernel,
    out_type=jax.ShapeDtypeStruct((NC, GN, D), jnp.float32),
    mesh=_MESH,
    scratch_types=[
        pltpu.VMEM((BLK, CHUNK), jnp.int32),
        pltpu.VMEM((BLK, CHUNK), jnp.int32),
        pltpu.VMEM((CHUNK, D), jnp.float32),
        pltpu.VMEM((CHUNK, D), jnp.float32),
        pltpu.VMEM_SHARED((GN, D), jnp.float32),
        pltpu.SemaphoreType.DMA,
        pltpu.SemaphoreType.DMA,
    ],
)
def _sc_aggregate(src3, dst3, u, S, idx_s, idx_d, buf0, buf1, acc, sem0, sem1):
    c = lax.axis_index("c")
    s = lax.axis_index("s")
    w = _wid()
    bufs = (buf0, buf1)
    sems = (sem0, sem1)
    def zrow(r, _):
        for col in range(D // L):
            buf0[r, pl.ds(col * L, L)] = _z16()
        return 0
    lax.fori_loop(0, CHUNK, zrow, 0)
    zbase = s * _ZROWS
    for kk in range(_ZROWS // CHUNK):
        pltpu.sync_copy(buf0, acc.at[pl.ds(zbase + kk * CHUNK, CHUNK)])
    if _ZROWS % CHUNK:
        pltpu.sync_copy(buf0.at[pl.ds(0, _ZROWS % CHUNK)],
                        acc.at[pl.ds(zbase + (_ZROWS // CHUNK) * CHUNK,
                                     _ZROWS % CHUNK)])
    plsc.subcore_barrier()

    def wait_gather(buf, sem):
        pltpu.make_async_copy(u.at[pl.ds(0, CHUNK)], buf, sem).wait()

    def blk_body(blk, _):
        pltpu.sync_copy(src3.at[w, pl.ds(blk * BLK, BLK)], idx_s)
        pltpu.sync_copy(dst3.at[w, pl.ds(blk * BLK, BLK)], idx_d)
        pltpu.async_copy(u.at[idx_s.at[0]], bufs[0], sems[0])
        for j in range(BLK):
            wait_gather(bufs[j % 2], sems[j % 2])
            if j + 1 < BLK:
                pltpu.async_copy(u.at[idx_s.at[j + 1]],
                                 bufs[(j + 1) % 2], sems[(j + 1) % 2])
            pltpu.sync_copy(bufs[j % 2], acc.at[idx_d.at[j]], add=True)
        return 0

    lax.fori_loop(0, NBLK, blk_body, 0)
    plsc.subcore_barrier()
    obase = s * _ZROWS
    for kk in range(_ZROWS // CHUNK):
        pltpu.sync_copy(acc.at[pl.ds(obase + kk * CHUNK, CHUNK)], buf0)
        pltpu.sync_copy(buf0, S.at[c, pl.ds(obase + kk * CHUNK, CHUNK)])
    rem2 = _ZROWS % CHUNK
    if rem2:
        tail = obase + (_ZROWS // CHUNK) * CHUNK
        pltpu.sync_copy(acc.at[pl.ds(tail, rem2)], buf0.at[pl.ds(0, rem2)])
        pltpu.sync_copy(buf0.at[pl.ds(0, rem2)], S.at[c, pl.ds(tail, rem2)])
    return


def _tc_mix_body(s0, s1, u, xw, dinv, b, hn_ref):
    agg = dinv[...] * (s0[...] + s1[...] + u[...])
    h = LAMDA * agg + (1.0 - LAMDA) * xw[...] + b[...]
    nrm = jnp.sqrt(jnp.sum(h * h, axis=1, keepdims=True))
    hn_ref[...] = h / jnp.maximum(nrm, 1e-12)


def _tc_mix(s0, s1, u, xw, dinv, b2):
    g = N // _R
    return pl.pallas_call(
        _tc_mix_body,
        grid=(g,),
        in_specs=[
            pl.BlockSpec((_R, D), lambda i: (i, 0)),
            pl.BlockSpec((_R, D), lambda i: (i, 0)),
            pl.BlockSpec((_R, D), lambda i: (i, 0)),
            pl.BlockSpec((_R, D), lambda i: (i, 0)),
            pl.BlockSpec((_R, 1), lambda i: (i, 0)),
            pl.BlockSpec((1, D), lambda i: (0, 0)),
        ],
        out_specs=pl.BlockSpec((_R, D), lambda i: (i, 0)),
        out_shape=jax.ShapeDtypeStruct((N, D), jnp.float32),
    )(s0, s1, u, xw, dinv, b2)


_BPW = B_IDX // NW


@functools.partial(
    pl.kernel,
    out_type=[
        jax.ShapeDtypeStruct((B_IDX, D), jnp.float32),
        jax.ShapeDtypeStruct((B_IDX, D), jnp.float32),
    ],
    mesh=_MESH,
    scratch_types=[
        pltpu.VMEM((_BPW,), jnp.int32),
        pltpu.VMEM((_BPW,), jnp.int32),
        pltpu.VMEM((_BPW, D), jnp.float32),
        pltpu.VMEM((_BPW, D), jnp.float32),
        pltpu.SemaphoreType.DMA,
        pltpu.SemaphoreType.DMA,
    ],
)
def _sc_lookup(hn, xi, yi, o1, o2, ix, iy, bx, by, sx, sy):
    w = _wid()
    base = w * _BPW
    pltpu.sync_copy(xi.at[pl.ds(base, _BPW)], ix)
    pltpu.sync_copy(yi.at[pl.ds(base, _BPW)], iy)
    cx = pltpu.async_copy(hn.at[ix], bx, sx)
    cy = pltpu.async_copy(hn.at[iy], by, sy)
    cx.wait()
    cy.wait()
    pltpu.sync_copy(bx, o1.at[pl.ds(base, _BPW)])
    pltpu.sync_copy(by, o2.at[pl.ds(base, _BPW)])


def kernel(x, y, edge_index, embedding, W, b):
    x = x.astype(jnp.int32)
    y = y.astype(jnp.int32)
    src = edge_index[0].astype(jnp.int32)
    dst = edge_index[1].astype(jnp.int32)

    pad = EP - E
    pi = jnp.arange(pad, dtype=jnp.int32)
    src_p = jnp.concatenate([src, pi % N]).reshape(NW, K, CHUNK)
    dst_p = jnp.concatenate([dst, N + pi % PAD_ROWS]).reshape(NW, K, CHUNK)

    degp = _sc_degree(dst_p)
    d0 = degp[:N].reshape(N, 1)
    d1 = degp[GN:GN + N].reshape(N, 1)

    xw, u, dinv = _tc_scale(embedding, W, d0, d1)

    S = _sc_aggregate(src_p, dst_p, u)

    hn = _tc_mix(S[0, :N], S[1, :N], u, xw, dinv, b.reshape(1, D))

    o1, o2 = _sc_lookup(hn, x, y)
    return (o1, o2)


# speedup vs baseline: 32.9015x; 1.0063x over previous
import functools

import jax
import jax.numpy as jnp
from jax import lax
from jax.experimental import pallas as pl
from jax.experimental.pallas import tpu as pltpu
from jax.experimental.pallas import tpu_sc as plsc

N = 10000
E = 320000
D = 128
B_IDX = 4096
LAMDA = 0.8

NC, NS, L = 2, 16, 16
NW = NC * NS
CHUNK = 128
BLK = 8
K = 80
NBLK = K // BLK
EP = NW * K * CHUNK
GN = 10112
PAD_ROWS = GN - N

_MESH = plsc.VectorSubcoreMesh(
    core_axis_name="c", subcore_axis_name="s", num_cores=NC, num_subcores=NS)

def _z16():
    return jnp.full((L,), 0.0, jnp.float32)


def _o16():
    return jnp.full((L,), 1.0, jnp.float32)


def _wid():
    return lax.axis_index("c") * NS + lax.axis_index("s")


@functools.partial(
    pl.kernel,
    out_type=jax.ShapeDtypeStruct((NC * GN,), jnp.float32),
    mesh=_MESH,
    scratch_types=[
        pltpu.VMEM((K, CHUNK), jnp.int32),
        pltpu.VMEM((640,), jnp.float32),
        pltpu.VMEM((CHUNK,), jnp.float32),
        pltpu.VMEM_SHARED((GN,), jnp.float32),
    ],
)
def _sc_degree(dst3, degp, idx_v, zbuf, ones_v, deg_sh):
    c = lax.axis_index("c")
    s = lax.axis_index("s")
    w = _wid()
    def fill_z(i, _):
        zbuf[pl.ds(i * L, L)] = _z16()
        return 0
    lax.fori_loop(0, 640 // L, fill_z, 0)
    def fill_o(i, _):
        ones_v[pl.ds(i * L, L)] = _o16()
        return 0
    lax.fori_loop(0, CHUNK // L, fill_o, 0)
    pltpu.sync_copy(zbuf.at[pl.ds(0, GN // NS)],
                    deg_sh.at[pl.ds(s * (GN // NS), GN // NS)])
    plsc.subcore_barrier()
    pltpu.sync_copy(dst3.at[w], idx_v)
    def chunk(j, _):
        pltpu.sync_copy(ones_v, deg_sh.at[idx_v.at[j]], add=True)
        return 0
    lax.fori_loop(0, K, chunk, 0)
    plsc.subcore_barrier()
    pltpu.sync_copy(deg_sh.at[pl.ds(s * (GN // NS), GN // NS)],
                    zbuf.at[pl.ds(0, GN // NS)])
    pltpu.sync_copy(zbuf.at[pl.ds(0, GN // NS)],
                    degp.at[pl.ds(c * GN + s * (GN // NS), GN // NS)])


_R = 1000


def _tc_matmul_body(emb, W, xw_ref):
    xw_ref[...] = jnp.dot(emb[...], W[...], preferred_element_type=jnp.float32)


def _tc_matmul(emb, W):
    return pl.pallas_call(
        _tc_matmul_body,
        grid=(N // _R,),
        in_specs=[
            pl.BlockSpec((_R, D), lambda i: (i, 0)),
            pl.BlockSpec((D, D), lambda i: (0, 0)),
        ],
        out_specs=pl.BlockSpec((_R, D), lambda i: (i, 0)),
        out_shape=jax.ShapeDtypeStruct((N, D), jnp.float32),
    )(emb, W)


def _tc_scale_body(xw, d0, d1, u_ref, dinv_ref):
    deg = d0[...] + d1[...] + 1.0
    dinv = lax.rsqrt(deg)
    u_ref[...] = xw[...] * dinv
    dinv_ref[...] = dinv


def _tc_scale(xw, d0, d1):
    return pl.pallas_call(
        _tc_scale_body,
        grid=(N // _R,),
        in_specs=[
            pl.BlockSpec((_R, D), lambda i: (i, 0)),
            pl.BlockSpec((_R, 1), lambda i: (i, 0)),
            pl.BlockSpec((_R, 1), lambda i: (i, 0)),
        ],
        out_specs=[
            pl.BlockSpec((_R, D), lambda i: (i, 0)),
            pl.BlockSpec((_R, 1), lambda i: (i, 0)),
        ],
        out_shape=[
            jax.ShapeDtypeStruct((N, D), jnp.float32),
            jax.ShapeDtypeStruct((N, 1), jnp.float32),
        ],
    )(xw, d0, d1)


_ZROWS = GN // NS
_OROWS = N // NS


@functools.partial(
    pl.kernel,
    out_type=jax.ShapeDtypeStruct((NC, GN, D), jnp.float32),
    mesh=_MESH,
    scratch_types=[
        pltpu.VMEM((BLK, CHUNK), jnp.int32),
        pltpu.VMEM((BLK, CHUNK), jnp.int32),
        pltpu.VMEM((CHUNK, D), jnp.float32),
        pltpu.VMEM((CHUNK, D), jnp.float32),
        pltpu.VMEM_SHARED((GN, D), jnp.float32),
        pltpu.SemaphoreType.DMA,
        pltpu.SemaphoreType.DMA,
        pltpu.SemaphoreType.DMA,
        pltpu.SemaphoreType.DMA,
    ],
)
def _sc_aggregate(src3, dst3, u, S, idx_s, idx_d, buf0, buf1, acc,
                  sem0, sem1, ssem0, ssem1):
    c = lax.axis_index("c")
    s = lax.axis_index("s")
    w = _wid()
    def zrow(r, _):
        for col in range(D // L):
            buf0[r, pl.ds(col * L, L)] = _z16()
        return 0
    lax.fori_loop(0, CHUNK, zrow, 0)
    zbase = s * _ZROWS
    for kk in range(_ZROWS // CHUNK):
        pltpu.sync_copy(buf0, acc.at[pl.ds(zbase + kk * CHUNK, CHUNK)])
    if _ZROWS % CHUNK:
        pltpu.sync_copy(buf0.at[pl.ds(0, _ZROWS % CHUNK)],
                        acc.at[pl.ds(zbase + (_ZROWS // CHUNK) * CHUNK,
                                     _ZROWS % CHUNK)])
    plsc.subcore_barrier()

    def wait_done(buf, sem):
        pltpu.make_async_copy(u.at[pl.ds(0, CHUNK)], buf, sem).wait()

    bufs = (buf0, buf1)
    gsems = (sem0, sem1)
    ssems = (ssem0, ssem1)

    def blk_body(blk, _):
        @pl.when(blk > 0)
        def _():
            wait_done(bufs[0], ssems[0])
            wait_done(bufs[1], ssems[1])
        pltpu.sync_copy(src3.at[w, pl.ds(blk * BLK, BLK)], idx_s)
        pltpu.sync_copy(dst3.at[w, pl.ds(blk * BLK, BLK)], idx_d)
        pltpu.async_copy(u.at[idx_s.at[0]], bufs[0], gsems[0])
        for j in range(BLK):
            wait_done(bufs[j % 2], gsems[j % 2])
            pltpu.async_copy(bufs[j % 2], acc.at[idx_d.at[j]], ssems[j % 2],
                             add=True)
            if j + 1 < BLK:
                if j >= 1:
                    wait_done(bufs[(j - 1) % 2], ssems[(j - 1) % 2])
                pltpu.async_copy(u.at[idx_s.at[j + 1]],
                                 bufs[(j + 1) % 2], gsems[(j + 1) % 2])
        return 0

    lax.fori_loop(0, NBLK, blk_body, 0)
    wait_done(bufs[0], ssems[0])
    wait_done(bufs[1], ssems[1])
    plsc.subcore_barrier()
    obase = s * _ZROWS
    for kk in range(_ZROWS // CHUNK):
        pltpu.sync_copy(acc.at[pl.ds(obase + kk * CHUNK, CHUNK)], buf0)
        pltpu.sync_copy(buf0, S.at[c, pl.ds(obase + kk * CHUNK, CHUNK)])
    rem2 = _ZROWS % CHUNK
    if rem2:
        tail = obase + (_ZROWS // CHUNK) * CHUNK
        pltpu.sync_copy(acc.at[pl.ds(tail, rem2)], buf0.at[pl.ds(0, rem2)])
        pltpu.sync_copy(buf0.at[pl.ds(0, rem2)], S.at[c, pl.ds(tail, rem2)])
    return


def _tc_mix_body(s0, s1, u, xw, dinv, b, hn_ref):
    agg = dinv[...] * (s0[...] + s1[...] + u[...])
    h = LAMDA * agg + (1.0 - LAMDA) * xw[...] + b[...]
    nrm = jnp.sqrt(jnp.sum(h * h, axis=1, keepdims=True))
    hn_ref[...] = h / jnp.maximum(nrm, 1e-12)


def _tc_mix(s0, s1, u, xw, dinv, b2):
    g = N // _R
    return pl.pallas_call(
        _tc_mix_body,
        grid=(g,),
        in_specs=[
            pl.BlockSpec((_R, D), lambda i: (i, 0)),
            pl.BlockSpec((_R, D), lambda i: (i, 0)),
            pl.BlockSpec((_R, D), lambda i: (i, 0)),
            pl.BlockSpec((_R, D), lambda i: (i, 0)),
            pl.BlockSpec((_R, 1), lambda i: (i, 0)),
            pl.BlockSpec((1, D), lambda i: (0, 0)),
        ],
        out_specs=pl.BlockSpec((_R, D), lambda i: (i, 0)),
        out_shape=jax.ShapeDtypeStruct((N, D), jnp.float32),
    )(s0, s1, u, xw, dinv, b2)


_BPW = B_IDX // NW


@functools.partial(
    pl.kernel,
    out_type=[
        jax.ShapeDtypeStruct((B_IDX, D), jnp.float32),
        jax.ShapeDtypeStruct((B_IDX, D), jnp.float32),
    ],
    mesh=_MESH,
    scratch_types=[
        pltpu.VMEM((_BPW,), jnp.int32),
        pltpu.VMEM((_BPW,), jnp.int32),
        pltpu.VMEM((_BPW, D), jnp.float32),
        pltpu.VMEM((_BPW, D), jnp.float32),
        pltpu.SemaphoreType.DMA,
        pltpu.SemaphoreType.DMA,
    ],
)
def _sc_lookup(hn, xi, yi, o1, o2, ix, iy, bx, by, sx, sy):
    w = _wid()
    base = w * _BPW
    pltpu.sync_copy(xi.at[pl.ds(base, _BPW)], ix)
    pltpu.sync_copy(yi.at[pl.ds(base, _BPW)], iy)
    cx = pltpu.async_copy(hn.at[ix], bx, sx)
    cy = pltpu.async_copy(hn.at[iy], by, sy)
    cx.wait()
    cy.wait()
    pltpu.sync_copy(bx, o1.at[pl.ds(base, _BPW)])
    pltpu.sync_copy(by, o2.at[pl.ds(base, _BPW)])


def kernel(x, y, edge_index, embedding, W, b):
    x = x.astype(jnp.int32)
    y = y.astype(jnp.int32)
    src = edge_index[0].astype(jnp.int32)
    dst = edge_index[1].astype(jnp.int32)

    pad = EP - E
    pi = jnp.arange(pad, dtype=jnp.int32)
    src_p = jnp.concatenate([src, pi % N]).reshape(NW, K, CHUNK)
    dst_p = jnp.concatenate([dst, N + pi % PAD_ROWS]).reshape(NW, K, CHUNK)

    xw = _tc_matmul(embedding, W)
    degp = _sc_degree(dst_p)
    d0 = degp[:N].reshape(N, 1)
    d1 = degp[GN:GN + N].reshape(N, 1)

    u, dinv = _tc_scale(xw, d0, d1)

    S = _sc_aggregate(src_p, dst_p, u)

    hn = _tc_mix(S[0, :N], S[1, :N], u, xw, dinv, b.reshape(1, D))

    o1, o2 = _sc_lookup(hn, x, y)
    return (o1, o2)


# speedup vs baseline: 38.7751x; 1.1785x over previous
import functools

import jax
import jax.numpy as jnp
from jax import lax
from jax.experimental import pallas as pl
from jax.experimental.pallas import tpu as pltpu
from jax.experimental.pallas import tpu_sc as plsc

N = 10000
E = 320000
D = 128
B_IDX = 4096
LAMDA = 0.8

NC, NS, L = 2, 16, 16
NW = NC * NS
CHUNK = 128
BLK = 8
K = 80
NBLK = K // BLK
EPT = K * CHUNK
EP = NW * EPT
GN = 10240
PAD_ROWS = GN - N
NWORD = GN // 32
TSLICE = GN // NS

_MESH = plsc.VectorSubcoreMesh(
    core_axis_name="c", subcore_axis_name="s", num_cores=NC, num_subcores=NS)


def _z16():
    return jnp.full((L,), 0.0, jnp.float32)


def _o16():
    return jnp.full((L,), 1.0, jnp.float32)


def _wid():
    return lax.axis_index("c") * NS + lax.axis_index("s")


@functools.partial(
    pl.kernel,
    out_type=jax.ShapeDtypeStruct((NC * GN,), jnp.float32),
    mesh=_MESH,
    scratch_types=[
        pltpu.VMEM((K, CHUNK), jnp.int32),
        pltpu.VMEM((TSLICE,), jnp.float32),
        pltpu.VMEM((CHUNK,), jnp.float32),
        pltpu.VMEM_SHARED((GN,), jnp.float32),
    ],
)
def _sc_degree(dst3, degp, idx_v, zbuf, ones_v, deg_sh):
    c = lax.axis_index("c")
    s = lax.axis_index("s")
    w = _wid()
    def fill_z(i, _):
        zbuf[pl.ds(i * L, L)] = _z16()
        return 0
    lax.fori_loop(0, TSLICE // L, fill_z, 0)
    def fill_o(i, _):
        ones_v[pl.ds(i * L, L)] = _o16()
        return 0
    lax.fori_loop(0, CHUNK // L, fill_o, 0)
    pltpu.sync_copy(zbuf, deg_sh.at[pl.ds(s * TSLICE, TSLICE)])
    plsc.subcore_barrier()
    pltpu.sync_copy(dst3.at[w], idx_v)
    def chunk(j, _):
        pltpu.sync_copy(ones_v, deg_sh.at[idx_v.at[j]], add=True)
        return 0
    lax.fori_loop(0, K, chunk, 0)
    plsc.subcore_barrier()
    pltpu.sync_copy(deg_sh.at[pl.ds(s * TSLICE, TSLICE)], zbuf)
    pltpu.sync_copy(zbuf, degp.at[pl.ds(c * GN + s * TSLICE, TSLICE)])


_R = 2000


def _tc_matmul_body(emb, W, xw_ref):
    xw_ref[...] = jnp.dot(emb[...], W[...], preferred_element_type=jnp.float32)


def _tc_matmul(emb, W):
    return pl.pallas_call(
        _tc_matmul_body,
        grid=(N // _R,),
        in_specs=[
            pl.BlockSpec((_R, D), lambda i: (i, 0)),
            pl.BlockSpec((D, D), lambda i: (0, 0)),
        ],
        out_specs=pl.BlockSpec((_R, D), lambda i: (i, 0)),
        out_shape=jax.ShapeDtypeStruct((N, D), jnp.float32),
    )(emb, W)


def _tc_scale_body(xw, d0, d1, u_ref, dinv_ref):
    deg = d0[...] + d1[...] + 1.0
    dinv = lax.rsqrt(deg)
    u_ref[...] = xw[...] * dinv
    dinv_ref[...] = dinv


def _tc_scale(xw, d0, d1):
    return pl.pallas_call(
        _tc_scale_body,
        grid=(N // _R,),
        in_specs=[
            pl.BlockSpec((_R, D), lambda i: (i, 0)),
            pl.BlockSpec((_R, 1), lambda i: (i, 0)),
            pl.BlockSpec((_R, 1), lambda i: (i, 0)),
        ],
        out_specs=[
            pl.BlockSpec((_R, D), lambda i: (i, 0)),
            pl.BlockSpec((_R, 1), lambda i: (i, 0)),
        ],
        out_shape=[
            jax.ShapeDtypeStruct((N, D), jnp.float32),
            jax.ShapeDtypeStruct((N, 1), jnp.float32),
        ],
    )(xw, d0, d1)


_OFULL = N // NS - 1
_SHIFT = 14
_MASKD = (1 << _SHIFT) - 1


@functools.partial(
    pl.kernel,
    out_type=jax.ShapeDtypeStruct((NC, N, D), jnp.float32),
    mesh=_MESH,
    scratch_types=[
        pltpu.VMEM((BLK, CHUNK), jnp.int32),
        pltpu.VMEM((BLK, CHUNK), jnp.int32),
        pltpu.VMEM((BLK, CHUNK), jnp.int32),
        pltpu.VMEM((EPT + CHUNK,), jnp.int32),
        pltpu.VMEM((NWORD,), jnp.int32),
        pltpu.VMEM((TSLICE,), jnp.float32),
        pltpu.VMEM((CHUNK,), jnp.float32),
        pltpu.VMEM((CHUNK, D), jnp.float32),
        pltpu.VMEM((CHUNK, D), jnp.float32),
        pltpu.VMEM_SHARED((GN, D), jnp.float32),
        pltpu.VMEM_SHARED((GN,), jnp.float32),
        pltpu.SemaphoreType.DMA,
        pltpu.SemaphoreType.DMA,
        pltpu.SemaphoreType.DMA,
        pltpu.SemaphoreType.DMA,
    ],
    compiler_params=pltpu.CompilerParams(needs_layout_passes=False),
)
def _sc_aggregate(src3, dst3, u, xi, yi, S,
                  idx_s, idx_d, uidx, comp, bmv, tmpc, ones_v, buf0, buf1,
                  acc, cnt, gsem0, gsem1, ssem0, ssem1):
    c = lax.axis_index("c")
    s = lax.axis_index("s")
    w = _wid()
    bufs = (buf0, buf1)
    gsems = (gsem0, gsem1)
    ssems = (ssem0, ssem1)

    def zrow(r, _):
        for col in range(D // L):
            buf0[r, pl.ds(col * L, L)] = _z16()
        return 0
    lax.fori_loop(0, CHUNK, zrow, 0)
    def fill_z1(i, _):
        tmpc[pl.ds(i * L, L)] = _z16()
        return 0
    lax.fori_loop(0, TSLICE // L, fill_z1, 0)
    def fill_o(i, _):
        ones_v[pl.ds(i * L, L)] = _o16()
        return 0
    lax.fori_loop(0, CHUNK // L, fill_o, 0)
    zbase = s * TSLICE
    for kk in range(TSLICE // CHUNK):
        pltpu.sync_copy(buf0, acc.at[pl.ds(zbase + kk * CHUNK, CHUNK)])
    pltpu.sync_copy(tmpc, cnt.at[pl.ds(zbase, TSLICE)])
    plsc.subcore_barrier()

    xbase = s * (B_IDX // NS)
    pltpu.sync_copy(xi.at[pl.ds(xbase, CHUNK)], uidx.at[0])
    pltpu.sync_copy(xi.at[pl.ds(xbase + CHUNK, CHUNK)], uidx.at[1])
    pltpu.sync_copy(yi.at[pl.ds(xbase, CHUNK)], uidx.at[2])
    pltpu.sync_copy(yi.at[pl.ds(xbase + CHUNK, CHUNK)], uidx.at[3])
    for r in range(4):
        pltpu.sync_copy(ones_v, cnt.at[uidx.at[r]], add=True)
    plsc.subcore_barrier()

    lanes = lax.iota(jnp.int32, L)
    zero16 = jnp.full((L,), 0, jnp.int32)
    one16 = jnp.full((L,), 1, jnp.int32)
    def bm_q(q, _):
        pltpu.sync_copy(cnt.at[pl.ds(q * 512, 512)], tmpc.at[pl.ds(0, 512)])
        def bm_bit(j, accv):
            vals = plsc.load_gather(tmpc, [lanes * 32 + j])
            return jnp.bitwise_or(
                accv, jnp.where(vals > 0.0, jnp.left_shift(one16, j), zero16))
        wordv = lax.fori_loop(0, 32, bm_bit, zero16)
        bmv[pl.ds(q * L, L)] = wordv
        return 0
    lax.fori_loop(0, GN // 512, bm_q, 0)

    def filt_blk(blk, off):
        pltpu.sync_copy(src3.at[w, pl.ds(blk * BLK, BLK)], idx_s)
        pltpu.sync_copy(dst3.at[w, pl.ds(blk * BLK, BLK)], idx_d)
        def filt_chunk(j, off2):
            for g in range(CHUNK // L):
                s16 = idx_s[j, pl.ds(g * L, L)]
                d16 = idx_d[j, pl.ds(g * L, L)]
                wv = plsc.load_gather(bmv, [jnp.right_shift(d16, 5)])
                bit = jnp.bitwise_and(
                    jnp.right_shift(wv, jnp.bitwise_and(d16, 31)), 1)
                m = bit > 0
                packed = jnp.bitwise_or(jnp.left_shift(s16, _SHIFT), d16)
                ranks = plsc.cumsum(m.astype(jnp.int32))
                plsc.store_scatter(comp, [off2 + ranks - 1], packed, mask=m)
                off2 = off2 + jnp.max(ranks)
            return off2
        return lax.fori_loop(0, BLK, filt_chunk, off)
    off = lax.fori_loop(0, NBLK, filt_blk, jnp.int32(0))

    for g in range(CHUNK // L):
        pad = jnp.bitwise_or(jnp.left_shift(lanes * 613, _SHIFT),
                             N + g * L + lanes)
        plsc.store_scatter(comp, [off + g * L + lanes], pad)
    nchunks = jnp.right_shift(off, 7) + 1

    def wait_done(buf, sem):
        pltpu.make_async_copy(u.at[pl.ds(0, CHUNK)], buf, sem).wait()

    def unpack(j, slot):
        for g in range(CHUNK // L):
            pk = comp[pl.ds(j * CHUNK + g * L, L)]
            uidx[2 * slot, pl.ds(g * L, L)] = jnp.right_shift(pk, _SHIFT)
            uidx[2 * slot + 1, pl.ds(g * L, L)] = jnp.bitwise_and(pk, _MASKD)

    def slot_body(j, b):
        wait_done(bufs[b], gsems[b])
        pltpu.async_copy(bufs[b], acc.at[uidx.at[2 * b + 1]], ssems[b],
                         add=True)
        @pl.when(j > 0)
        def _():
            wait_done(bufs[1 - b], ssems[1 - b])
        @pl.when(j + 1 < nchunks)
        def _():
            unpack(j + 1, 1 - b)
            pltpu.async_copy(u.at[uidx.at[2 * (1 - b)]],
                             bufs[1 - b], gsems[1 - b])

    unpack(0, 0)
    pltpu.async_copy(u.at[uidx.at[0]], bufs[0], gsems[0])

    def pair_body(p, _):
        slot_body(2 * p, 0)
        @pl.when(2 * p + 1 < nchunks)
        def _():
            slot_body(2 * p + 1, 1)
        return 0
    lax.fori_loop(0, (nchunks + 1) // 2, pair_body, 0)
    @pl.when(jnp.bitwise_and(nchunks, 1) == 1)
    def _():
        wait_done(bufs[0], ssems[0])
    @pl.when(jnp.bitwise_and(nchunks, 1) == 0)
    def _():
        wait_done(bufs[1], ssems[1])
    plsc.subcore_barrier()

    @pl.when(s < NS - 1)
    def _():
        obase = s * _OFULL
        for kk in range(_OFULL // CHUNK):
            pltpu.sync_copy(acc.at[pl.ds(obase + kk * CHUNK, CHUNK)], buf0)
            pltpu.sync_copy(buf0, S.at[c, pl.ds(obase + kk * CHUNK, CHUNK)])
        rem = _OFULL % CHUNK
        tail = obase + (_OFULL // CHUNK) * CHUNK
        pltpu.sync_copy(acc.at[pl.ds(tail, rem)], buf0.at[pl.ds(0, rem)])
        pltpu.sync_copy(buf0.at[pl.ds(0, rem)], S.at[c, pl.ds(tail, rem)])
    @pl.when(s == NS - 1)
    def _():
        obase = (NS - 1) * _OFULL
        for kk in range((N - (NS - 1) * _OFULL) // CHUNK):
            pltpu.sync_copy(acc.at[pl.ds(obase + kk * CHUNK, CHUNK)], buf0)
            pltpu.sync_copy(buf0, S.at[c, pl.ds(obase + kk * CHUNK, CHUNK)])
    return


def _tc_mix_body(s0, s1, u, xw, dinv, b, hn_ref):
    agg = dinv[...] * (s0[0] + s1[0] + u[...])
    h = LAMDA * agg + (1.0 - LAMDA) * xw[...] + b[...]
    nrm = jnp.sqrt(jnp.sum(h * h, axis=1, keepdims=True))
    hn_ref[...] = h / jnp.maximum(nrm, 1e-12)


def _tc_mix(S, u, xw, dinv, b2):
    return pl.pallas_call(
        _tc_mix_body,
        grid=(N // _R,),
        in_specs=[
            pl.BlockSpec((1, _R, D), lambda i: (0, i, 0)),
            pl.BlockSpec((1, _R, D), lambda i: (1, i, 0)),
            pl.BlockSpec((_R, D), lambda i: (i, 0)),
            pl.BlockSpec((_R, D), lambda i: (i, 0)),
            pl.BlockSpec((_R, 1), lambda i: (i, 0)),
            pl.BlockSpec((1, D), lambda i: (0, 0)),
        ],
        out_specs=pl.BlockSpec((_R, D), lambda i: (i, 0)),
        out_shape=jax.ShapeDtypeStruct((N, D), jnp.float32),
    )(S, S, u, xw, dinv, b2)


_BPW = B_IDX // NW


@functools.partial(
    pl.kernel,
    out_type=[
        jax.ShapeDtypeStruct((B_IDX, D), jnp.float32),
        jax.ShapeDtypeStruct((B_IDX, D), jnp.float32),
    ],
    mesh=_MESH,
    scratch_types=[
        pltpu.VMEM((_BPW,), jnp.int32),
        pltpu.VMEM((_BPW,), jnp.int32),
        pltpu.VMEM((_BPW, D), jnp.float32),
        pltpu.VMEM((_BPW, D), jnp.float32),
        pltpu.SemaphoreType.DMA,
        pltpu.SemaphoreType.DMA,
    ],
)
def _sc_lookup(hn, xi, yi, o1, o2, ix, iy, bx, by, sx, sy):
    w = _wid()
    base = w * _BPW
    pltpu.sync_copy(xi.at[pl.ds(base, _BPW)], ix)
    pltpu.sync_copy(yi.at[pl.ds(base, _BPW)], iy)
    cx = pltpu.async_copy(hn.at[ix], bx, sx)
    cy = pltpu.async_copy(hn.at[iy], by, sy)
    cx.wait()
    cy.wait()
    pltpu.sync_copy(bx, o1.at[pl.ds(base, _BPW)])
    pltpu.sync_copy(by, o2.at[pl.ds(base, _BPW)])


def kernel(x, y, edge_index, embedding, W, b):
    x = x.astype(jnp.int32)
    y = y.astype(jnp.int32)
    src = edge_index[0].astype(jnp.int32)
    dst = edge_index[1].astype(jnp.int32)

    pad = EP - E
    pi = jnp.arange(pad, dtype=jnp.int32)
    src_p = jnp.concatenate([src, pi % N]).reshape(NW, K, CHUNK)
    dst_p = jnp.concatenate([dst, N + pi % PAD_ROWS]).reshape(NW, K, CHUNK)

    xw = _tc_matmul(embedding, W)
    degp = _sc_degree(dst_p)
    d0 = degp[:N].reshape(N, 1)
    d1 = degp[GN:GN + N].reshape(N, 1)

    u, dinv = _tc_scale(xw, d0, d1)

    S = _sc_aggregate(src_p, dst_p, u, x, y)

    hn = _tc_mix(S, u, xw, dinv, b.reshape(1, D))

    o1, o2 = _sc_lookup(hn, x, y)
    return (o1, o2)
